# baseline (speedup 1.0000x reference)
import functools

import jax
import jax.numpy as jnp
from jax.experimental import pallas as pl
from jax.experimental.pallas import tpu as pltpu

_N = 100000
_E = 3200000
_WIDTH = 128
_DEPTH = 3

_BLK = 1024


def _seg_body(blk, a_ref, b_ref, d_ref, w_ref, o_ref, cvec_ref, cdst_ref):
    i = pl.program_id(0)

    @pl.when(i == 0)
    def _():
        cvec_ref[...] = jnp.full((1, _WIDTH), jnp.inf, jnp.float32)
        cdst_ref[...] = jnp.full((1, 1), -1, jnp.int32)

    w = w_ref[...]
    u = w[0:1, :]
    v = w[1:2, :]
    c = w[2:3, :]
    w2 = w[3:4, :]
    b2 = w[4:5, 0:1]
    def rbf(t):
        return t.astype(jnp.bfloat16).astype(jnp.float32)

    a = rbf(a_ref[...])
    b = rbf(b_ref[...])
    d = d_ref[...]
    z = a * rbf(u) + b * rbf(v)

    off = 1
    while off < blk:
        zs = jnp.concatenate(
            [jnp.full((off, _WIDTH), jnp.inf, jnp.float32), z[:-off, :]], axis=0)
        ds = jnp.concatenate(
            [jnp.full((off, 1), -1, jnp.int32), d[:-off, :]], axis=0)
        z = jnp.where(ds == d, jnp.minimum(z, zs), z)
        off *= 2

    z = jnp.where(d == cdst_ref[...], jnp.minimum(z, cvec_ref[...]), z)
    cvec_ref[...] = z[blk - 1:blk, :]
    cdst_ref[...] = d[blk - 1:blk, :]

    agg = jax.nn.relu(z + c)
    agg = agg.astype(jnp.bfloat16).astype(jnp.float32)
    o_ref[...] = jax.nn.relu(jnp.sum(agg * rbf(w2), axis=1, keepdims=True) + b2)


def _layer(a_s, attr_s, dst_s, wb, e, blk):
    grid = (e // blk,)
    return pl.pallas_call(
        functools.partial(_seg_body, blk),
        grid=grid,
        in_specs=[
            pl.BlockSpec((blk, 1), lambda i: (i, 0)),
            pl.BlockSpec((blk, 1), lambda i: (i, 0)),
            pl.BlockSpec((blk, 1), lambda i: (i, 0)),
            pl.BlockSpec((8, _WIDTH), lambda i: (0, 0)),
        ],
        out_specs=pl.BlockSpec((blk, 1), lambda i: (i, 0)),
        out_shape=jax.ShapeDtypeStruct((e, 1), jnp.float32),
        scratch_shapes=[
            pltpu.VMEM((1, _WIDTH), jnp.float32),
            pltpu.VMEM((1, 1), jnp.int32),
        ],
    )(a_s, attr_s, dst_s, wb)


def _pack_weights(w1, b1, w2, b2):
    wb = jnp.zeros((8, _WIDTH), jnp.float32)
    wb = wb.at[0, :].set(w1[:, 0])
    wb = wb.at[1, :].set(w1[:, 1])
    wb = wb.at[2, :].set(b1)
    wb = wb.at[3, :].set(w2[0, :])
    wb = wb.at[4, :].set(b2[0])
    return wb


def kernel(x, edge_index, edge_attr, W1s, b1s, W2s, b2s):
    src = edge_index[0]
    dst = edge_index[1]
    attr = edge_attr[:, 0]
    dst_s, src_s, attr_s = jax.lax.sort((dst, src, attr), num_keys=1)
    ar = jnp.arange(_N, dtype=dst.dtype)
    rt = jnp.searchsorted(dst_s, ar, side='right')
    lt = jnp.searchsorted(dst_s, ar, side='left')
    has_in = rt > lt
    p = jnp.clip(rt - 1, 0, _E - 1).astype(jnp.int32)
    p_src_s = p[src_s]
    hs_src_s = has_in[src_s]

    dst_s2 = dst_s.astype(jnp.int32)[:, None]
    attr_s2 = attr_s[:, None]
    a = x[:, 0][src_s]
    res = None
    for l in range(_DEPTH):
        wb = _pack_weights(W1s[l], b1s[l], W2s[l], b2s[l])
        out_val = _layer(a[:, None], attr_s2, dst_s2, wb, _E, _BLK)[:, 0]
        iso = jax.nn.relu(b2s[l][0])
        if l < _DEPTH - 1:
            a = jnp.where(hs_src_s, out_val[p_src_s], iso)
        else:
            res = jnp.where(has_in, out_val[p], iso)[:, None]
    return res


# speedup vs baseline: 3.9254x; 3.9254x over previous
import functools

import jax
import jax.numpy as jnp
from jax import lax
from jax.experimental import pallas as pl
from jax.experimental.pallas import tpu as pltpu, tpu_sc as plsc

_N = 100000
_E = 3200000
_WIDTH = 128
_DEPTH = 3

_BLK = 1024

_NC, _NS = 2, 16
_NW = _NC * _NS
_NPAD = 100352


def _sc_gather(table, idx, total, chunk, nchunks, out_dtype):
    mesh = plsc.VectorSubcoreMesh(core_axis_name="c", subcore_axis_name="s")

    @functools.partial(
        pl.kernel, mesh=mesh,
        out_type=jax.ShapeDtypeStruct((total,), out_dtype),
        scratch_types=[
            pltpu.VMEM((chunk,), jnp.int32),
            pltpu.VMEM((chunk,), out_dtype),
            pltpu.SemaphoreType.DMA,
        ],
    )
    def k(table_hbm, idx_hbm, out_hbm, idx_v, rows_v, sem):
        wid = lax.axis_index("s") * _NC + lax.axis_index("c")
        base = wid * (chunk * nchunks)

        def step(j, carry):
            off = base + j * chunk
            pltpu.sync_copy(idx_hbm.at[pl.ds(off, chunk)], idx_v)
            pltpu.async_copy(table_hbm.at[idx_v], rows_v, sem).wait()
            pltpu.sync_copy(rows_v, out_hbm.at[pl.ds(off, chunk)])
            return carry

        lax.fori_loop(0, nchunks, step, 0)

    return k(table, idx)


def _seg_body(blk, a_ref, b_ref, d_ref, w_ref, o_ref, cvec_ref, cdst_ref):
    i = pl.program_id(0)

    @pl.when(i == 0)
    def _():
        cvec_ref[...] = jnp.full((1, _WIDTH), jnp.inf, jnp.float32)
        cdst_ref[...] = jnp.full((1, 1), -1, jnp.int32)

    w = w_ref[...]
    u = w[0:1, :]
    v = w[1:2, :]
    c = w[2:3, :]
    w2 = w[3:4, :]
    b2 = w[4:5, 0:1]
    def rbf(t):
        return t.astype(jnp.bfloat16).astype(jnp.float32)

    a = rbf(a_ref[...])
    b = rbf(b_ref[...])
    d = d_ref[...]
    z = a * rbf(u) + b * rbf(v)

    off = 1
    while off < blk:
        zs = jnp.concatenate(
            [jnp.full((off, _WIDTH), jnp.inf, jnp.float32), z[:-off, :]], axis=0)
        ds = jnp.concatenate(
            [jnp.full((off, 1), -1, jnp.int32), d[:-off, :]], axis=0)
        z = jnp.where(ds == d, jnp.minimum(z, zs), z)
        off *= 2

    z = jnp.where(d == cdst_ref[...], jnp.minimum(z, cvec_ref[...]), z)
    cvec_ref[...] = z[blk - 1:blk, :]
    cdst_ref[...] = d[blk - 1:blk, :]

    agg = jax.nn.relu(z + c)
    agg = agg.astype(jnp.bfloat16).astype(jnp.float32)
    o_ref[...] = jax.nn.relu(jnp.sum(agg * rbf(w2), axis=1, keepdims=True) + b2)


def _layer(a_s, attr_s, dst_s, wb, e, blk):
    grid = (e // blk,)
    return pl.pallas_call(
        functools.partial(_seg_body, blk),
        grid=grid,
        in_specs=[
            pl.BlockSpec((blk, 1), lambda i: (i, 0)),
            pl.BlockSpec((blk, 1), lambda i: (i, 0)),
            pl.BlockSpec((blk, 1), lambda i: (i, 0)),
            pl.BlockSpec((8, _WIDTH), lambda i: (0, 0)),
        ],
        out_specs=pl.BlockSpec((blk, 1), lambda i: (i, 0)),
        out_shape=jax.ShapeDtypeStruct((e, 1), jnp.float32),
        scratch_shapes=[
            pltpu.VMEM((1, _WIDTH), jnp.float32),
            pltpu.VMEM((1, 1), jnp.int32),
        ],
    )(a_s, attr_s, dst_s, wb)


def _pack_weights(w1, b1, w2, b2):
    wb = jnp.zeros((8, _WIDTH), jnp.float32)
    wb = wb.at[0, :].set(w1[:, 0])
    wb = wb.at[1, :].set(w1[:, 1])
    wb = wb.at[2, :].set(b1)
    wb = wb.at[3, :].set(w2[0, :])
    wb = wb.at[4, :].set(b2[0])
    return wb


def kernel(x, edge_index, edge_attr, W1s, b1s, W2s, b2s):
    src = edge_index[0]
    dst = edge_index[1]
    attr = edge_attr[:, 0]
    dst_s, src_s, attr_s = jax.lax.sort((dst, src, attr), num_keys=1)
    ar = jnp.arange(_N, dtype=dst.dtype)
    rt = jnp.searchsorted(dst_s, ar, side='right')
    lt = jnp.searchsorted(dst_s, ar, side='left')
    has_in = rt > lt
    p = jnp.clip(rt - 1, 0, _E - 1).astype(jnp.int32)

    ph = p * 2 + has_in.astype(jnp.int32)
    ph_s = _sc_gather(ph, src_s, _E, 10000, 10, jnp.int32)
    p_src_s = ph_s >> 1
    hs_src_s = (ph_s & 1) == 1

    dst_s2 = dst_s.astype(jnp.int32)[:, None]
    attr_s2 = attr_s[:, None]
    a = _sc_gather(x[:, 0], src_s, _E, 10000, 10, jnp.float32)
    res = None
    for l in range(_DEPTH):
        wb = _pack_weights(W1s[l], b1s[l], W2s[l], b2s[l])
        out_val = _layer(a[:, None], attr_s2, dst_s2, wb, _E, _BLK)[:, 0]
        iso = jax.nn.relu(b2s[l][0])
        if l < _DEPTH - 1:
            g = _sc_gather(out_val, p_src_s, _E, 10000, 10, jnp.float32)
            a = jnp.where(hs_src_s, g, iso)
        else:
            p_pad = jnp.pad(p, (0, _NPAD - _N))
            outf = _sc_gather(out_val, p_pad, _NPAD, 3136, 1, jnp.float32)[:_N]
            res = jnp.where(has_in, outf, iso)[:, None]
    return res


# speedup vs baseline: 4.6189x; 1.1767x over previous
import functools

import jax
import jax.numpy as jnp
from jax import lax
from jax.experimental import pallas as pl
from jax.experimental.pallas import tpu as pltpu, tpu_sc as plsc

_N = 100000
_E = 3200000
_WIDTH = 128
_DEPTH = 3

_BLK = 1024

_NC, _NS = 2, 16
_NW = _NC * _NS
_NPAD = 100352


def _sc_gather(table, idx, total, chunk, nchunks, out_dtype):
    mesh = plsc.VectorSubcoreMesh(core_axis_name="c", subcore_axis_name="s")

    @functools.partial(
        pl.kernel, mesh=mesh,
        out_type=jax.ShapeDtypeStruct((total,), out_dtype),
        scratch_types=[
            pltpu.VMEM((chunk,), jnp.int32),
            pltpu.VMEM((chunk,), out_dtype),
            pltpu.SemaphoreType.DMA,
        ],
    )
    def k(table_hbm, idx_hbm, out_hbm, idx_v, rows_v, sem):
        wid = lax.axis_index("s") * _NC + lax.axis_index("c")
        base = wid * (chunk * nchunks)

        def step(j, carry):
            off = base + j * chunk
            pltpu.sync_copy(idx_hbm.at[pl.ds(off, chunk)], idx_v)
            pltpu.async_copy(table_hbm.at[idx_v], rows_v, sem).wait()
            pltpu.sync_copy(rows_v, out_hbm.at[pl.ds(off, chunk)])
            return carry

        lax.fori_loop(0, nchunks, step, 0)

    return k(table, idx)


def _seg_body(blk, a_ref, b_ref, d_ref, h_ref, w_ref, o_ref, cvec_ref, cdst_ref):
    i = pl.program_id(0)

    @pl.when(i == 0)
    def _():
        cvec_ref[...] = jnp.full((_WIDTH, 1), jnp.inf, jnp.float32)
        cdst_ref[...] = jnp.full((1, 1), -1, jnp.int32)

    w = w_ref[...]
    u = w[:, 0:1]
    v = w[:, 1:2]
    c = w[:, 2:3]
    w2 = w[:, 3:4]
    b2 = w[0:1, 4:5]
    iso = w[0:1, 5:6]

    def rbf(t):
        return t.astype(jnp.bfloat16).astype(jnp.float32)

    hs = h_ref[0]
    a = hs * a_ref[0] + (1.0 - hs) * iso
    a = rbf(a)
    b = rbf(b_ref[0])
    d = d_ref[0]
    z = rbf(u) * a + rbf(v) * b

    off = 1
    while off < blk:
        zs = jnp.concatenate(
            [jnp.full((_WIDTH, off), jnp.inf, jnp.float32), z[:, :-off]], axis=1)
        ds = jnp.concatenate(
            [jnp.full((1, off), -1, jnp.int32), d[:, :-off]], axis=1)
        z = jnp.where(ds == d, jnp.minimum(z, zs), z)
        off *= 2

    z = jnp.where(d == cdst_ref[...], jnp.minimum(z, cvec_ref[...]), z)
    cvec_ref[...] = z[:, blk - 1:blk]
    cdst_ref[...] = d[:, blk - 1:blk]

    agg = jax.nn.relu(z + c)
    agg = agg.astype(jnp.bfloat16).astype(jnp.float32)
    o_ref[0] = jax.nn.relu(jnp.sum(agg * rbf(w2), axis=0, keepdims=True) + b2)


def _layer(a_s, attr3, dst3, hs3, wb, e, blk):
    nb = e // blk
    out = pl.pallas_call(
        functools.partial(_seg_body, blk),
        grid=(nb,),
        in_specs=[
            pl.BlockSpec((1, 1, blk), lambda i: (i, 0, 0)),
            pl.BlockSpec((1, 1, blk), lambda i: (i, 0, 0)),
            pl.BlockSpec((1, 1, blk), lambda i: (i, 0, 0)),
            pl.BlockSpec((1, 1, blk), lambda i: (i, 0, 0)),
            pl.BlockSpec((_WIDTH, 128), lambda i: (0, 0)),
        ],
        out_specs=pl.BlockSpec((1, 1, blk), lambda i: (i, 0, 0)),
        out_shape=jax.ShapeDtypeStruct((nb, 1, blk), jnp.float32),
        scratch_shapes=[
            pltpu.VMEM((_WIDTH, 1), jnp.float32),
            pltpu.VMEM((1, 1), jnp.int32),
        ],
    )(a_s.reshape(nb, 1, blk), attr3, dst3, hs3, wb)
    return out.reshape(e)


def _pack_weights(w1, b1, w2, b2, iso):
    wb = jnp.zeros((_WIDTH, 128), jnp.float32)
    wb = wb.at[:, 0].set(w1[:, 0])
    wb = wb.at[:, 1].set(w1[:, 1])
    wb = wb.at[:, 2].set(b1)
    wb = wb.at[:, 3].set(w2[0, :])
    wb = wb.at[0, 4].set(b2[0])
    wb = wb.at[0, 5].set(iso)
    return wb


def kernel(x, edge_index, edge_attr, W1s, b1s, W2s, b2s):
    src = edge_index[0]
    dst = edge_index[1]
    attr = edge_attr[:, 0]
    dst_s, src_s, attr_s = jax.lax.sort((dst, src, attr), num_keys=1)

    pos = jnp.full((_N,), -1, jnp.int32).at[dst_s].max(
        jnp.arange(_E, dtype=jnp.int32))
    has_in = pos >= 0
    p = jnp.maximum(pos, 0)

    ph = p * 2 + has_in.astype(jnp.int32)
    ph_s = _sc_gather(ph, src_s, _E, 10000, 10, jnp.int32)
    p_src_s = ph_s >> 1
    hs_f = (ph_s & 1).astype(jnp.float32)

    nb = _E // _BLK
    dst3 = dst_s.astype(jnp.int32).reshape(nb, 1, _BLK)
    attr3 = attr_s.reshape(nb, 1, _BLK)
    hs3 = hs_f.reshape(nb, 1, _BLK)
    ones3 = jnp.ones((nb, 1, _BLK), jnp.float32)

    a = _sc_gather(x[:, 0], src_s, _E, 10000, 10, jnp.float32)
    res = None
    for l in range(_DEPTH):
        iso_prev = jax.nn.relu(b2s[l - 1][0]) if l > 0 else 0.0
        wb = _pack_weights(W1s[l], b1s[l], W2s[l], b2s[l], iso_prev)
        h3 = ones3 if l == 0 else hs3
        out_val = _layer(a, attr3, dst3, h3, wb, _E, _BLK)
        iso = jax.nn.relu(b2s[l][0])
        if l < _DEPTH - 1:
            a = _sc_gather(out_val, p_src_s, _E, 10000, 10, jnp.float32)
        else:
            p_pad = jnp.pad(p, (0, _NPAD - _N))
            outf = _sc_gather(out_val, p_pad, _NPAD, 3136, 1, jnp.float32)[:_N]
            res = jnp.where(has_in, outf, iso)[:, None]
    return res
